# baseline (speedup 1.0000x reference)
import functools

import jax
import jax.numpy as jnp
from jax import lax
from jax.experimental import pallas as pl
from jax.experimental.pallas import tpu as pltpu
from jax.experimental.pallas import tpu_sc as plsc

_D = 128
_GC = 40
_NSLOT = 3
_NC = 2
_NS = 16
_NW = _NC * _NS
_IHALF = 5000


@functools.partial(jax.jit, static_argnames=("n_edges",))
def _link_embed(src_idx, dst_idx, table, n_edges):
    w_edges = n_edges // _NW
    n_groups = w_edges // _GC
    assert n_edges % (_NW * _GC) == 0 and w_edges == 2 * _IHALF
    assert n_groups % _NSLOT == 1
    t_iters = n_groups // _NSLOT
    half_groups = _IHALF // _GC

    mesh = plsc.VectorSubcoreMesh(
        core_axis_name="c", subcore_axis_name="s",
        num_cores=_NC, num_subcores=_NS,
    )

    n_rows = table.shape[0]
    assert n_rows % (_NS * 8) == 0
    rows_per_tile = n_rows // _NS

    @functools.partial(
        pl.kernel,
        out_type=jax.ShapeDtypeStruct((n_edges, 2 * _D), jnp.float32),
        mesh=mesh,
        scratch_types=[
            pltpu.VMEM((_IHALF,), jnp.int32),
            pltpu.VMEM((_IHALF,), jnp.int32),
            pltpu.VMEM((_NSLOT, 2, _GC, _D), jnp.float32),
            pltpu.VMEM_SHARED((n_rows, _D), jnp.float32),
            [pltpu.SemaphoreType.DMA] * _NSLOT,
            [pltpu.SemaphoreType.DMA] * _NSLOT,
        ],
    )
    def run(src_hbm, dst_hbm, table_hbm, out_hbm,
            src_v, dst_v, rows_v, table_sh, gsems, wsems):
        sid = lax.axis_index("s")
        wid = lax.axis_index("c") * _NS + sid
        edge_base = wid * w_edges
        r0 = sid * rows_per_tile
        pltpu.sync_copy(table_hbm.at[pl.ds(r0, rows_per_tile)],
                        table_sh.at[pl.ds(r0, rows_per_tile)])

        def load_idx_half(h):
            e0 = edge_base + h * _IHALF
            pltpu.sync_copy(src_hbm.at[pl.ds(e0, _IHALF)], src_v)
            pltpu.sync_copy(dst_hbm.at[pl.ds(e0, _IHALF)], dst_v)

        load_idx_half(0)
        plsc.subcore_barrier()

        def start_gathers(g, p):
            off = lax.rem(g * _GC, _IHALF)
            pltpu.async_copy(table_sh.at[src_v.at[pl.ds(off, _GC)]],
                             rows_v.at[p, 0], gsems[p])
            pltpu.async_copy(table_sh.at[dst_v.at[pl.ds(off, _GC)]],
                             rows_v.at[p, 1], gsems[p])

        def wait_gathers(p):
            for h in range(2):
                pltpu.make_async_copy(table_hbm.at[pl.ds(0, _GC)],
                                      rows_v.at[p, h], gsems[p]).wait()

        def start_write(g, p):
            e0 = edge_base + g * _GC
            pltpu.async_copy(rows_v.at[p, 0],
                             out_hbm.at[pl.ds(e0, _GC), pl.ds(0, _D)],
                             wsems[p])
            pltpu.async_copy(rows_v.at[p, 1],
                             out_hbm.at[pl.ds(e0, _GC), pl.ds(_D, _D)],
                             wsems[p])

        def wait_write(p):
            for h in range(2):
                pltpu.make_async_copy(table_hbm.at[pl.ds(0, _GC)],
                                      rows_v.at[p, h], wsems[p]).wait()

        def body(j, carry):
            for q in range(_NSLOT):
                g = _NSLOT * j + q
                pm1 = (q - 1) % _NSLOT

                def ab():
                    wait_gathers(pm1)
                    start_write(g - 1, pm1)

                def c():
                    wait_write(q)

                if q == 0:
                    pl.when(j > 0)(ab)
                    pl.when(j > 0)(c)
                elif q == 1:
                    ab()
                    pl.when(j > 0)(c)
                else:
                    ab()
                    pl.when(j > 0)(c)
                    pl.when(g == half_groups)(lambda: load_idx_half(1))
                start_gathers(g, q)
            return carry

        lax.fori_loop(0, t_iters, body, 0)

        g_last = n_groups - 1
        p_last = g_last % _NSLOT
        pm1 = (g_last - 1) % _NSLOT
        wait_gathers(pm1)
        start_write(g_last - 1, pm1)
        wait_write(p_last)
        start_gathers(g_last, p_last)
        wait_gathers(p_last)
        start_write(g_last, p_last)
        wait_write((g_last + 1) % _NSLOT)
        wait_write(pm1)
        wait_write(p_last)

    return run(src_idx, dst_idx, table)


def kernel(X_2, indices):
    E = indices.shape[0]
    idx32 = indices.astype(jnp.int32)
    pad = (-X_2.shape[0]) % (_NS * 8)
    table = jnp.pad(X_2, ((0, pad), (0, 0))) if pad else X_2
    return _link_embed(idx32[:, 0], idx32[:, 1], table, E)


# speedup vs baseline: 1.0016x; 1.0016x over previous
import functools

import jax
import jax.numpy as jnp
from jax import lax
from jax.experimental import pallas as pl
from jax.experimental.pallas import tpu as pltpu
from jax.experimental.pallas import tpu_sc as plsc

_D = 128
_GC = 40
_NSLOT = 3
_NC = 2
_NS = 16
_NW = _NC * _NS
_IHALF = 5000


@functools.partial(jax.jit, static_argnames=("n_edges",))
def _link_embed(src_idx, dst_idx, table, n_edges):
    w_edges = n_edges // _NW
    n_groups = w_edges // _GC
    half_groups = _IHALF // _GC
    assert n_edges % (_NW * _GC) == 0 and w_edges == 2 * _IHALF
    assert n_groups % _NSLOT == 1 and n_groups > 2 * _NSLOT
    assert half_groups % _NSLOT == _NSLOT - 1
    t_iters = n_groups // _NSLOT

    mesh = plsc.VectorSubcoreMesh(
        core_axis_name="c", subcore_axis_name="s",
        num_cores=_NC, num_subcores=_NS,
    )

    n_rows = table.shape[0]
    assert n_rows % (_NS * 8) == 0
    rows_per_tile = n_rows // _NS

    @functools.partial(
        pl.kernel,
        out_type=jax.ShapeDtypeStruct((n_edges, 2 * _D), jnp.float32),
        mesh=mesh,
        scratch_types=[
            pltpu.VMEM((_IHALF,), jnp.int32),
            pltpu.VMEM((_IHALF,), jnp.int32),
            pltpu.VMEM((_NSLOT, 2 * _GC, _D), jnp.float32),
            pltpu.VMEM_SHARED((n_rows, _D), jnp.float32),
            [pltpu.SemaphoreType.DMA] * _NSLOT,
            [pltpu.SemaphoreType.DMA] * _NSLOT,
        ],
    )
    def run(src_hbm, dst_hbm, table_hbm, out_hbm,
            src_v, dst_v, rows_v, table_sh, gsems, wsems):
        sid = lax.axis_index("s")
        wid = lax.axis_index("c") * _NS + sid
        edge_base = wid * w_edges
        r0 = sid * rows_per_tile
        pltpu.sync_copy(table_hbm.at[pl.ds(r0, rows_per_tile)],
                        table_sh.at[pl.ds(r0, rows_per_tile)])

        def load_idx_half(h):
            e0 = edge_base + h * _IHALF
            pltpu.sync_copy(src_hbm.at[pl.ds(e0, _IHALF)], src_v)
            pltpu.sync_copy(dst_hbm.at[pl.ds(e0, _IHALF)], dst_v)

        load_idx_half(0)
        plsc.subcore_barrier()

        def start_gathers(g, p):
            off = lax.rem(g * _GC, _IHALF)
            pltpu.async_copy(table_sh.at[src_v.at[pl.ds(off, _GC)]],
                             rows_v.at[p, pl.ds(0, _GC)], gsems[p])
            pltpu.async_copy(table_sh.at[dst_v.at[pl.ds(off, _GC)]],
                             rows_v.at[p, pl.ds(_GC, _GC)], gsems[p])

        def wait_gathers(p):
            pltpu.make_async_copy(table_hbm.at[pl.ds(0, 2 * _GC)],
                                  rows_v.at[p], gsems[p]).wait()

        def start_write(g, p):
            e0 = edge_base + g * _GC
            pltpu.async_copy(rows_v.at[p, pl.ds(0, _GC)],
                             out_hbm.at[pl.ds(e0, _GC), pl.ds(0, _D)],
                             wsems[p])
            pltpu.async_copy(rows_v.at[p, pl.ds(_GC, _GC)],
                             out_hbm.at[pl.ds(e0, _GC), pl.ds(_D, _D)],
                             wsems[p])

        def wait_write(p):
            pltpu.make_async_copy(table_hbm.at[pl.ds(0, 2 * _GC)],
                                  rows_v.at[p], wsems[p]).wait()

        start_gathers(0, 0)
        wait_gathers(0)
        start_write(0, 0)
        start_gathers(1, 1)
        wait_gathers(1)
        start_write(1, 1)
        start_gathers(2, 2)

        def body(j, carry):
            for q in range(_NSLOT):
                g = _NSLOT * j + q
                pm1 = (q - 1) % _NSLOT
                wait_gathers(pm1)
                start_write(g - 1, pm1)
                wait_write(q)
                if q == _NSLOT - 1:
                    pl.when(g == half_groups)(lambda: load_idx_half(1))
                start_gathers(g, q)
            return carry

        lax.fori_loop(1, t_iters, body, 0)

        g_last = n_groups - 1
        p_last = g_last % _NSLOT
        pm1 = (g_last - 1) % _NSLOT
        wait_gathers(pm1)
        start_write(g_last - 1, pm1)
        wait_write(p_last)
        start_gathers(g_last, p_last)
        wait_gathers(p_last)
        start_write(g_last, p_last)
        wait_write((g_last + 1) % _NSLOT)
        wait_write(pm1)
        wait_write(p_last)

    return run(src_idx, dst_idx, table)


def kernel(X_2, indices):
    E = indices.shape[0]
    idx32 = indices.astype(jnp.int32)
    pad = (-X_2.shape[0]) % (_NS * 8)
    table = jnp.pad(X_2, ((0, pad), (0, 0))) if pad else X_2
    return _link_embed(idx32[:, 0], idx32[:, 1], table, E)


# speedup vs baseline: 1.0657x; 1.0640x over previous
import functools

import jax
import jax.numpy as jnp
from jax import lax
from jax.experimental import pallas as pl
from jax.experimental.pallas import tpu as pltpu
from jax.experimental.pallas import tpu_sc as plsc

_D = 128
_GC = 40
_NC = 2
_NS = 16
_NW = _NC * _NS


@functools.partial(jax.jit, static_argnames=("n_edges",))
def _link_embed(src_idx, dst_idx, table, n_edges):
    assert n_edges % (_NW * 2 * _GC) == 0
    w_edges = n_edges // _NW
    n_groups = w_edges // _GC
    t_iters = n_groups // 2

    mesh = plsc.VectorSubcoreMesh(
        core_axis_name="c", subcore_axis_name="s",
        num_cores=_NC, num_subcores=_NS,
    )

    n_rows = table.shape[0]
    assert n_rows % 8 == 0
    stripe = ((n_rows + _NS - 1) // _NS + 7) // 8 * 8
    last_stripe = n_rows - (_NS - 1) * stripe
    assert 0 < last_stripe <= stripe

    @functools.partial(
        pl.kernel,
        out_type=jax.ShapeDtypeStruct((n_edges, 2 * _D), jnp.float32),
        mesh=mesh,
        scratch_types=[
            pltpu.VMEM((w_edges,), jnp.int32),
            pltpu.VMEM((w_edges,), jnp.int32),
            pltpu.VMEM((2, 2 * _GC, _D), jnp.float32),
            pltpu.VMEM_SHARED((n_rows, _D), jnp.float32),
            pltpu.SemaphoreType.DMA,
            pltpu.SemaphoreType.DMA,
        ],
    )
    def run(src_hbm, dst_hbm, table_hbm, out_hbm,
            src_v, dst_v, rows_v, table_sh, gsem0, gsem1):
        sid = lax.axis_index("s")
        wid = lax.axis_index("c") * _NS + sid
        edge_base = wid * w_edges
        r0 = sid * stripe

        @pl.when(sid < _NS - 1)
        def _():
            pltpu.sync_copy(table_hbm.at[pl.ds(r0, stripe)],
                            table_sh.at[pl.ds(r0, stripe)])

        @pl.when(sid == _NS - 1)
        def _():
            pltpu.sync_copy(table_hbm.at[pl.ds(r0, last_stripe)],
                            table_sh.at[pl.ds(r0, last_stripe)])

        pltpu.sync_copy(src_hbm.at[pl.ds(edge_base, w_edges)], src_v)
        pltpu.sync_copy(dst_hbm.at[pl.ds(edge_base, w_edges)], dst_v)
        plsc.subcore_barrier()

        def start_group(g, p, sem):
            off = g * _GC
            pltpu.async_copy(table_sh.at[src_v.at[pl.ds(off, _GC)]],
                             rows_v.at[p, pl.ds(0, _GC)], sem)
            pltpu.async_copy(table_sh.at[dst_v.at[pl.ds(off, _GC)]],
                             rows_v.at[p, pl.ds(_GC, _GC)], sem)

        def wait_group(p, sem):
            pltpu.make_async_copy(table_hbm.at[pl.ds(0, 2 * _GC)],
                                  rows_v.at[p], sem).wait()

        def write_group(g, p):
            e0 = edge_base + g * _GC
            pltpu.sync_copy(rows_v.at[p, pl.ds(0, _GC)],
                            out_hbm.at[pl.ds(e0, _GC), pl.ds(0, _D)])
            pltpu.sync_copy(rows_v.at[p, pl.ds(_GC, _GC)],
                            out_hbm.at[pl.ds(e0, _GC), pl.ds(_D, _D)])

        start_group(0, 0, gsem0)

        def body(j, carry):
            g0 = 2 * j
            start_group(g0 + 1, 1, gsem1)
            wait_group(0, gsem0)
            write_group(g0, 0)

            @pl.when(j < t_iters - 1)
            def _():
                start_group(g0 + 2, 0, gsem0)

            wait_group(1, gsem1)
            write_group(g0 + 1, 1)
            return carry

        lax.fori_loop(0, t_iters, body, 0)

    return run(src_idx, dst_idx, table)


def kernel(X_2, indices):
    E = indices.shape[0]
    idx32 = indices.astype(jnp.int32)
    return _link_embed(idx32[:, 0], idx32[:, 1], X_2, E)


# speedup vs baseline: 1.0878x; 1.0207x over previous
import functools

import jax
import jax.numpy as jnp
from jax import lax
from jax.experimental import pallas as pl
from jax.experimental.pallas import tpu as pltpu
from jax.experimental.pallas import tpu_sc as plsc

_D = 128
_GC = 80
_L = 16
_NC = 2
_NS = 16
_NW = _NC * _NS
_ICHUNK = 2000


@functools.partial(jax.jit, static_argnames=("n_edges",))
def _link_embed(packed_idx, table, n_edges):
    assert n_edges % (_NW * _GC) == 0
    w_edges = n_edges // _NW
    n_groups = w_edges // _GC
    assert n_groups % 2 == 1
    t_iters = n_groups // 2
    assert _ICHUNK % _GC == 0 and w_edges % _ICHUNK == 0
    chunk_groups = _ICHUNK // _GC

    mesh = plsc.VectorSubcoreMesh(
        core_axis_name="c", subcore_axis_name="s",
        num_cores=_NC, num_subcores=_NS,
    )

    n_rows = table.shape[0]
    assert n_rows % 8 == 0 and n_rows <= 2**16
    stripe = ((n_rows + _NS - 1) // _NS + 7) // 8 * 8
    last_stripe = n_rows - (_NS - 1) * stripe
    assert 0 < last_stripe <= stripe

    @functools.partial(
        pl.kernel,
        out_type=jax.ShapeDtypeStruct((n_edges, 2 * _D), jnp.float32),
        mesh=mesh,
        scratch_types=[
            pltpu.VMEM((_ICHUNK,), jnp.int32),
            pltpu.VMEM((2, 2, _GC), jnp.int32),
            pltpu.VMEM((2, 2 * _GC, _D), jnp.float32),
            pltpu.VMEM_SHARED((n_rows, _D), jnp.float32),
            pltpu.SemaphoreType.DMA,
            pltpu.SemaphoreType.DMA,
        ],
    )
    def run(pk_hbm, table_hbm, out_hbm,
            pk_v, idx_v, rows_v, table_sh, gsem0, gsem1):
        sid = lax.axis_index("s")
        wid = lax.axis_index("c") * _NS + sid
        edge_base = wid * w_edges
        r0 = sid * stripe

        @pl.when(sid < _NS - 1)
        def _():
            pltpu.sync_copy(table_hbm.at[pl.ds(r0, stripe)],
                            table_sh.at[pl.ds(r0, stripe)])

        @pl.when(sid == _NS - 1)
        def _():
            pltpu.sync_copy(table_hbm.at[pl.ds(r0, last_stripe)],
                            table_sh.at[pl.ds(r0, last_stripe)])

        def load_idx_chunk(c):
            pltpu.sync_copy(
                pk_hbm.at[pl.ds(edge_base + c * _ICHUNK, _ICHUNK)], pk_v)

        load_idx_chunk(0)
        plsc.subcore_barrier()

        def start_group(g, p, sem):
            loff = lax.rem(g * _GC, _ICHUNK)
            for k in range(_GC // _L):
                v = pk_v[pl.ds(loff + k * _L, _L)]
                idx_v[p, 0, pl.ds(k * _L, _L)] = lax.bitwise_and(v, 0xFFFF)
                idx_v[p, 1, pl.ds(k * _L, _L)] = lax.shift_right_logical(v, 16)
            pltpu.async_copy(table_sh.at[idx_v.at[p, 0]],
                             rows_v.at[p, pl.ds(0, _GC)], sem)
            pltpu.async_copy(table_sh.at[idx_v.at[p, 1]],
                             rows_v.at[p, pl.ds(_GC, _GC)], sem)

        def wait_group(p, sem):
            pltpu.make_async_copy(table_hbm.at[pl.ds(0, 2 * _GC)],
                                  rows_v.at[p], sem).wait()

        def write_group(g, p):
            e0 = edge_base + g * _GC
            pltpu.sync_copy(rows_v.at[p, pl.ds(0, _GC)],
                            out_hbm.at[pl.ds(e0, _GC), pl.ds(0, _D)])
            pltpu.sync_copy(rows_v.at[p, pl.ds(_GC, _GC)],
                            out_hbm.at[pl.ds(e0, _GC), pl.ds(_D, _D)])

        start_group(0, 0, gsem0)

        def reload_before(g):
            pl.when(lax.rem(g, chunk_groups) == 0)(
                lambda: load_idx_chunk(g // chunk_groups))

        def body(j, carry):
            g0 = 2 * j
            reload_before(g0 + 1)
            start_group(g0 + 1, 1, gsem1)
            wait_group(0, gsem0)
            write_group(g0, 0)
            reload_before(g0 + 2)
            start_group(g0 + 2, 0, gsem0)
            wait_group(1, gsem1)
            write_group(g0 + 1, 1)
            return carry

        lax.fori_loop(0, t_iters, body, 0)

        wait_group(0, gsem0)
        write_group(n_groups - 1, 0)

    return run(packed_idx, table)


def kernel(X_2, indices):
    E = indices.shape[0]
    idx32 = indices.astype(jnp.int32)
    packed = idx32[:, 0] | (idx32[:, 1] << 16)
    return _link_embed(packed, X_2, E)
